# baseline (speedup 1.0000x reference)
import functools

import jax
import jax.numpy as jnp
from jax import lax
from jax.experimental import pallas as pl
from jax.experimental.pallas import tpu as pltpu
from jax.experimental.pallas import tpu_sc as plsc

_EMB = 64
_NC = 2
_NS = 16
_NW = _NC * _NS
_G = 128


def _prep_table(table, w1s, bias, blk):
    V = table.shape[0]
    tT = table.T
    blk = min(V, blk)
    grid = ((V + blk - 1) // blk,)
    h = blk // 2

    def body(x, w, b, o):
        y = x[...].T
        y = jnp.dot(y, w[...], preferred_element_type=jnp.float32) + b[...]
        z = jnp.concatenate([y[:h], y[h:]], axis=1)
        o[...] = z.reshape(blk * _EMB)

    out = pl.pallas_call(
        body,
        grid=grid,
        in_specs=[pl.BlockSpec((_EMB, blk), lambda i: (0, i)),
                  pl.BlockSpec((_EMB, _EMB), lambda i: (0, 0)),
                  pl.BlockSpec((1, _EMB), lambda i: (0, 0))],
        out_specs=pl.BlockSpec((blk * _EMB,), lambda i: (i,)),
        out_shape=jax.ShapeDtypeStruct((grid[0] * blk * _EMB,), jnp.float32),
    )(tT, w1s, bias)
    return out, blk


def _prep_table_pad(table, w1s, bias, blk):
    V = table.shape[0]
    tT = table.T
    blk = min(V, blk)
    grid = ((V + blk - 1) // blk,)

    def body(x, w, b, o):
        y = x[...].T
        y = jnp.dot(y, w[...], preferred_element_type=jnp.float32) + b[...]
        z = jnp.concatenate([y, jnp.zeros((blk, _EMB), jnp.float32)], axis=1)
        o[...] = z.reshape(blk * 2 * _EMB)

    out = pl.pallas_call(
        body,
        grid=grid,
        in_specs=[pl.BlockSpec((_EMB, blk), lambda i: (0, i)),
                  pl.BlockSpec((_EMB, _EMB), lambda i: (0, 0)),
                  pl.BlockSpec((1, _EMB), lambda i: (0, 0))],
        out_specs=pl.BlockSpec((blk * 2 * _EMB,), lambda i: (i,)),
        out_shape=jax.ShapeDtypeStruct((V * 2 * _EMB,), jnp.float32),
    )(tT, w1s, bias)
    return out


def _pack_idx(r, blk):
    h = blk // 2
    b = r // blk
    j = r - b * blk
    jh = j // h
    return b * blk + 2 * (j - h * jh) + jh


def _sc_gather(idx2d, tables, n_tok):
    nt = len(tables)
    ng = n_tok // (_NW * _G)
    tpw = n_tok // _NW

    mesh = plsc.VectorSubcoreMesh(core_axis_name="c", subcore_axis_name="s",
                                  num_cores=_NC, num_subcores=_NS)

    @functools.partial(
        pl.kernel,
        out_type=tuple(jax.ShapeDtypeStruct((n_tok, _EMB), jnp.float32)
                       for _ in range(nt)),
        mesh=mesh,
        scratch_types=[
            pltpu.VMEM((nt, ng, _G), jnp.int32),
            pltpu.VMEM((nt, _G, _EMB), jnp.float32),
            pltpu.VMEM((nt, _G, _EMB), jnp.float32),
            pltpu.SemaphoreType.DMA,
            pltpu.SemaphoreType.DMA,
        ],
        compiler_params=pltpu.CompilerParams(use_tc_tiling_on_sc=False),
    )
    def k(idx_hbm, *rest):
        tabs = rest[:nt]
        outs = rest[nt:2 * nt]
        idx_v, rows_a, rows_b, sem_a, sem_b = rest[2 * nt:]
        wid = lax.axis_index("s") * _NC + lax.axis_index("c")
        tbase = wid * tpw
        for t in range(nt):
            pltpu.sync_copy(idx_hbm.at[t, wid], idx_v.at[t])

        def fire(g, buf, sem):
            return [pltpu.async_copy(tabs[t].at[idx_v.at[t, g]],
                                     buf.at[t], sem)
                    for t in range(nt)]

        fire(0, rows_a, sem_a)

        def pair2(i, carry):
            g0 = i * 2

            cps_b = fire(g0 + 1, rows_b, sem_b)
            for t in range(nt):
                pltpu.make_async_copy(tabs[t].at[idx_v.at[t, g0]],
                                      rows_a.at[t], sem_a).wait()
            for t in range(nt):
                pltpu.sync_copy(rows_a.at[t],
                                outs[t].at[pl.ds(tbase + g0 * _G, _G)])

            @pl.when(i < ng // 2 - 1)
            def _():
                fire(g0 + 2, rows_a, sem_a)

            for cp in cps_b:
                cp.wait()
            for t in range(nt):
                pltpu.sync_copy(rows_b.at[t],
                                outs[t].at[pl.ds(tbase + (g0 + 1) * _G, _G)])
            return carry

        lax.fori_loop(0, ng // 2, pair2, 0)

    return k(idx2d, *tables)


def _sc_gather_sum(idx2d, tables, n_tok):
    nt = len(tables)
    ng = n_tok // (_NW * _G)
    tpw = n_tok // _NW

    mesh = plsc.VectorSubcoreMesh(core_axis_name="c", subcore_axis_name="s",
                                  num_cores=_NC, num_subcores=_NS)

    @functools.partial(
        pl.kernel,
        out_type=jax.ShapeDtypeStruct((n_tok, _EMB), jnp.float32),
        mesh=mesh,
        scratch_types=[
            pltpu.VMEM((nt, ng, _G), jnp.int32),
            pltpu.VMEM((nt, _G, _EMB), jnp.float32),
            pltpu.VMEM((nt, _G, _EMB), jnp.float32),
            pltpu.VMEM((_G, _EMB), jnp.float32),
            pltpu.VMEM((_G, _EMB), jnp.float32),
            pltpu.SemaphoreType.DMA,
            pltpu.SemaphoreType.DMA,
        ],
        compiler_params=pltpu.CompilerParams(use_tc_tiling_on_sc=False),
    )
    def k(idx_hbm, *rest):
        tabs = rest[:nt]
        (out_s, idx_v, rows_a, rows_b, sum_a, sum_b,
         sem_a, sem_b) = rest[nt:]
        wid = lax.axis_index("s") * _NC + lax.axis_index("c")
        tbase = wid * tpw
        for t in range(nt):
            pltpu.sync_copy(idx_hbm.at[t, wid], idx_v.at[t])

        def fire(g, buf, sem):
            return [pltpu.async_copy(tabs[t].at[idx_v.at[t, g]],
                                     buf.at[t], sem)
                    for t in range(nt)]

        def drain(g, buf, sem):
            for t in range(nt):
                pltpu.make_async_copy(tabs[t].at[idx_v.at[t, g]],
                                      buf.at[t], sem).wait()

        def vsum(buf, sbuf):
            def add1(v, carry):
                r = v // 4
                c = pl.multiple_of((v % 4) * 16, 16)
                x = buf[0, r, pl.ds(c, 16)]
                for t in range(1, nt):
                    x = x + buf[t, r, pl.ds(c, 16)]
                sbuf[r, pl.ds(c, 16)] = x
                return carry
            lax.fori_loop(0, _G * 4, add1, 0)

        fire(0, rows_a, sem_a)

        def pair2(i, carry):
            g0 = i * 2
            cps_b = fire(g0 + 1, rows_b, sem_b)
            drain(g0, rows_a, sem_a)
            vsum(rows_a, sum_a)

            @pl.when(i < ng // 2 - 1)
            def _():
                fire(g0 + 2, rows_a, sem_a)

            pltpu.sync_copy(sum_a, out_s.at[pl.ds(tbase + g0 * _G, _G)])
            for cp in cps_b:
                cp.wait()
            vsum(rows_b, sum_b)
            pltpu.sync_copy(sum_b, out_s.at[pl.ds(tbase + (g0 + 1) * _G, _G)])
            return carry

        lax.fori_loop(0, ng // 2, pair2, 0)

    return k(idx2d, *tables)


def _tc_mlp(rows128, W2, b2, gamma, beta, n_half, blk):
    grid = (n_half // blk,)
    nr = len(rows128)

    def half(h, w2, b2r, gm, bt):
        h2 = jnp.dot(h, w2, preferred_element_type=jnp.float32) + b2r
        mu = jnp.mean(h2, axis=-1, keepdims=True)
        var = jnp.mean((h2 - mu) ** 2, axis=-1, keepdims=True)
        return (h2 - mu) / jnp.sqrt(var + 1e-3) * gm + bt

    def body(*refs):
        xs = refs[:nr]
        w2, b2r, gm, bt, o = refs[nr:]
        s = xs[0][...]
        for x in xs[1:]:
            s = s + x[...]
        h = jnp.maximum(s, 0.0)
        args = (w2[...], b2r[...], gm[...], bt[...])
        re = half(h[:, 0:64], *args)
        ro = half(h[:, 64:128], *args)
        o[...] = jnp.stack([re, ro], axis=1).reshape(2 * blk, _EMB)

    tok_spec = pl.BlockSpec((blk, 2 * _EMB), lambda i: (i, 0))
    out_spec = pl.BlockSpec((2 * blk, _EMB), lambda i: (i, 0))
    full = lambda shape: pl.BlockSpec(shape, lambda i: tuple(0 for _ in shape))
    return pl.pallas_call(
        body,
        grid=grid,
        in_specs=[tok_spec] * nr + [
            full((_EMB, _EMB)), full((1, _EMB)),
            full((1, _EMB)), full((1, _EMB)),
        ],
        out_specs=out_spec,
        out_shape=jax.ShapeDtypeStruct((2 * n_half, _EMB), jnp.float32),
    )(*rows128, W2, b2.reshape(1, _EMB),
      gamma.reshape(1, _EMB), beta.reshape(1, _EMB))


def kernel(video_ids, categories, tags, durations, timestamps,
           video_table, category_table, tag_table, duration_table, time_table,
           W1, b1, W2, b2, gamma, beta):
    B, L = video_ids.shape
    n_tok = B * L
    ng = n_tok // (_NW * _G)
    dur_buckets = (durations / 300.0 * 100.0).astype(jnp.int32)
    time_buckets = (timestamps % 168).astype(jnp.int32)
    zb = jnp.zeros((1, _EMB), jnp.float32)
    cat1, cblk = _prep_table(category_table, W1[64:128], b1.reshape(1, _EMB),
                             2048)
    tag1, gblk = _prep_table(tag_table, W1[128:192], zb, 4096)
    dur1, dblk = _prep_table(duration_table, W1[192:256], zb, 2048)
    tim1, tblk = _prep_table(time_table, W1[256:320], zb, 2048)
    idx_small = jnp.stack([
        _pack_idx(categories.reshape(-1).astype(jnp.int32), cblk),
        _pack_idx(tags.reshape(-1).astype(jnp.int32), gblk),
        _pack_idx(dur_buckets.reshape(-1), dblk),
        _pack_idx(time_buckets.reshape(-1), tblk),
    ]).reshape(4, _NW, ng, _G)
    vt_b, cat1, tag1, dur1, tim1 = jax.lax.optimization_barrier(
        (video_table, cat1, tag1, dur1, tim1))
    s_rows = _sc_gather_sum(
        idx_small,
        [a.reshape(a.shape[0] // _EMB, _EMB)
         for a in (cat1, tag1, dur1, tim1)], n_tok)
    vid1 = _prep_table_pad(vt_b, W1[0:64], zb, 8192)
    idx_video = (video_ids.reshape(-1).astype(jnp.int32)
                 * 2).reshape(1, _NW, ng, _G)
    (v_rows,) = _sc_gather(
        idx_video, [vid1.reshape(vid1.shape[0] // _EMB, _EMB)], n_tok)

    rows128 = [r.reshape(n_tok // 2, 2 * _EMB) for r in (v_rows, s_rows)]
    out = _tc_mlp(rows128, W2, b2, gamma, beta, n_tok // 2, blk=2048)
    return out.reshape(B, L, _EMB)


# speedup vs baseline: 1.0734x; 1.0734x over previous
import functools

import jax
import jax.numpy as jnp
from jax import lax
from jax.experimental import pallas as pl
from jax.experimental.pallas import tpu as pltpu
from jax.experimental.pallas import tpu_sc as plsc

_EMB = 64
_NC = 2
_NS = 16
_NW = _NC * _NS
_G = 128


def _prep_table_pad(table, w1s, bias, blk):
    V = table.shape[0]
    tT = table.T
    blk = min(V, blk)
    grid = ((V + blk - 1) // blk,)

    def body(x, w, b, o):
        y = x[...].T
        y = jnp.dot(y, w[...], preferred_element_type=jnp.float32) + b[...]
        z = jnp.concatenate([y, jnp.zeros((blk, _EMB), jnp.float32)], axis=1)
        o[...] = z.reshape(blk * 2 * _EMB)

    out = pl.pallas_call(
        body,
        grid=grid,
        in_specs=[pl.BlockSpec((_EMB, blk), lambda i: (0, i)),
                  pl.BlockSpec((_EMB, _EMB), lambda i: (0, 0)),
                  pl.BlockSpec((1, _EMB), lambda i: (0, 0))],
        out_specs=pl.BlockSpec((blk * 2 * _EMB,), lambda i: (i,)),
        out_shape=jax.ShapeDtypeStruct((V * 2 * _EMB,), jnp.float32),
    )(tT, w1s, bias)
    return out


def _sc_gather(idx2d, tables, n_tok):
    nt = len(tables)
    ng = n_tok // (_NW * _G)
    tpw = n_tok // _NW

    mesh = plsc.VectorSubcoreMesh(core_axis_name="c", subcore_axis_name="s",
                                  num_cores=_NC, num_subcores=_NS)

    @functools.partial(
        pl.kernel,
        out_type=tuple(jax.ShapeDtypeStruct((n_tok, _EMB), jnp.float32)
                       for _ in range(nt)),
        mesh=mesh,
        scratch_types=[
            pltpu.VMEM((nt, ng, _G), jnp.int32),
            pltpu.VMEM((nt, _G, _EMB), jnp.float32),
            pltpu.VMEM((nt, _G, _EMB), jnp.float32),
            pltpu.SemaphoreType.DMA,
            pltpu.SemaphoreType.DMA,
        ],
        compiler_params=pltpu.CompilerParams(use_tc_tiling_on_sc=False),
    )
    def k(idx_hbm, *rest):
        tabs = rest[:nt]
        outs = rest[nt:2 * nt]
        idx_v, rows_a, rows_b, sem_a, sem_b = rest[2 * nt:]
        wid = lax.axis_index("s") * _NC + lax.axis_index("c")
        tbase = wid * tpw
        for t in range(nt):
            pltpu.sync_copy(idx_hbm.at[t, wid], idx_v.at[t])

        def fire(g, buf, sem):
            return [pltpu.async_copy(
                        tabs[t].at[idx_v.at[t, g, pl.ds(k * 64, 64)]],
                        buf.at[t, pl.ds(k * 64, 64)], sem)
                    for t in range(nt) for k in range(2)]

        fire(0, rows_a, sem_a)

        def pair2(i, carry):
            g0 = i * 2

            cps_b = fire(g0 + 1, rows_b, sem_b)
            for t in range(nt):
                for k in range(2):
                    pltpu.make_async_copy(
                        tabs[t].at[idx_v.at[t, g0, pl.ds(k * 64, 64)]],
                        rows_a.at[t, pl.ds(k * 64, 64)], sem_a).wait()
            for t in range(nt):
                pltpu.sync_copy(rows_a.at[t],
                                outs[t].at[pl.ds(tbase + g0 * _G, _G)])

            @pl.when(i < ng // 2 - 1)
            def _():
                fire(g0 + 2, rows_a, sem_a)

            for cp in cps_b:
                cp.wait()
            for t in range(nt):
                pltpu.sync_copy(rows_b.at[t],
                                outs[t].at[pl.ds(tbase + (g0 + 1) * _G, _G)])
            return carry

        lax.fori_loop(0, ng // 2, pair2, 0)

    return k(idx2d, *tables)


def _sc_gather_sum(idx2d, tables, n_tok):
    nt = len(tables)
    ng = n_tok // (_NW * _G)
    tpw = n_tok // _NW

    mesh = plsc.VectorSubcoreMesh(core_axis_name="c", subcore_axis_name="s",
                                  num_cores=_NC, num_subcores=_NS)

    @functools.partial(
        pl.kernel,
        out_type=jax.ShapeDtypeStruct((n_tok, _EMB), jnp.float32),
        mesh=mesh,
        scratch_types=[
            pltpu.VMEM((nt, ng, _G), jnp.int32),
            pltpu.VMEM((nt, _G, _EMB), jnp.float32),
            pltpu.VMEM((nt, _G, _EMB), jnp.float32),
            pltpu.VMEM((_G, _EMB), jnp.float32),
            pltpu.VMEM((_G, _EMB), jnp.float32),
            pltpu.SemaphoreType.DMA,
            pltpu.SemaphoreType.DMA,
        ],
        compiler_params=pltpu.CompilerParams(use_tc_tiling_on_sc=False),
    )
    def k(idx_hbm, *rest):
        tabs = rest[:nt]
        (out_s, idx_v, rows_a, rows_b, sum_a, sum_b,
         sem_a, sem_b) = rest[nt:]
        wid = lax.axis_index("s") * _NC + lax.axis_index("c")
        tbase = wid * tpw
        for t in range(nt):
            pltpu.sync_copy(idx_hbm.at[t, wid], idx_v.at[t])

        def fire(g, buf, sem):
            return [pltpu.async_copy(tabs[t].at[idx_v.at[t, g]],
                                     buf.at[t], sem)
                    for t in range(nt)]

        def drain(g, buf, sem):
            for t in range(nt):
                pltpu.make_async_copy(tabs[t].at[idx_v.at[t, g]],
                                      buf.at[t], sem).wait()

        def vsum(buf, sbuf):
            def add1(v, carry):
                r = v // 4
                c = pl.multiple_of((v % 4) * 16, 16)
                x = buf[0, r, pl.ds(c, 16)]
                for t in range(1, nt):
                    x = x + buf[t, r, pl.ds(c, 16)]
                sbuf[r, pl.ds(c, 16)] = x
                return carry
            lax.fori_loop(0, _G * 4, add1, 0)

        fire(0, rows_a, sem_a)

        def pair2(i, carry):
            g0 = i * 2
            cps_b = fire(g0 + 1, rows_b, sem_b)
            drain(g0, rows_a, sem_a)
            vsum(rows_a, sum_a)

            @pl.when(i < ng // 2 - 1)
            def _():
                fire(g0 + 2, rows_a, sem_a)

            pltpu.sync_copy(sum_a, out_s.at[pl.ds(tbase + g0 * _G, _G)])
            for cp in cps_b:
                cp.wait()
            vsum(rows_b, sum_b)
            pltpu.sync_copy(sum_b, out_s.at[pl.ds(tbase + (g0 + 1) * _G, _G)])
            return carry

        lax.fori_loop(0, ng // 2, pair2, 0)

    return k(idx2d, *tables)


def _tc_mlp(rows128, W2, b2, gamma, beta, n_half, blk):
    grid = (n_half // blk,)
    nr = len(rows128)

    def half(h, w2, b2r, gm, bt):
        h2 = jnp.dot(h, w2, preferred_element_type=jnp.float32) + b2r
        mu = jnp.mean(h2, axis=-1, keepdims=True)
        var = jnp.mean((h2 - mu) ** 2, axis=-1, keepdims=True)
        return (h2 - mu) / jnp.sqrt(var + 1e-3) * gm + bt

    def body(*refs):
        xs = refs[:nr]
        w2, b2r, gm, bt, o = refs[nr:]
        s = xs[0][...]
        for x in xs[1:]:
            s = s + x[...]
        h = jnp.maximum(s, 0.0)
        args = (w2[...], b2r[...], gm[...], bt[...])
        re = half(h[:, 0:64], *args)
        ro = half(h[:, 64:128], *args)
        o[...] = jnp.stack([re, ro], axis=1).reshape(2 * blk, _EMB)

    tok_spec = pl.BlockSpec((blk, 2 * _EMB), lambda i: (i, 0))
    out_spec = pl.BlockSpec((2 * blk, _EMB), lambda i: (i, 0))
    full = lambda shape: pl.BlockSpec(shape, lambda i: tuple(0 for _ in shape))
    return pl.pallas_call(
        body,
        grid=grid,
        in_specs=[tok_spec] * nr + [
            full((_EMB, _EMB)), full((1, _EMB)),
            full((1, _EMB)), full((1, _EMB)),
        ],
        out_specs=out_spec,
        out_shape=jax.ShapeDtypeStruct((2 * n_half, _EMB), jnp.float32),
    )(*rows128, W2, b2.reshape(1, _EMB),
      gamma.reshape(1, _EMB), beta.reshape(1, _EMB))


def kernel(video_ids, categories, tags, durations, timestamps,
           video_table, category_table, tag_table, duration_table, time_table,
           W1, b1, W2, b2, gamma, beta):
    B, L = video_ids.shape
    n_tok = B * L
    ng = n_tok // (_NW * _G)
    dur_buckets = (durations / 300.0 * 100.0).astype(jnp.int32)
    time_buckets = (timestamps % 168).astype(jnp.int32)
    zb = jnp.zeros((1, _EMB), jnp.float32)
    cat1 = _prep_table_pad(category_table, W1[64:128], b1.reshape(1, _EMB),
                           2048)
    tag1 = _prep_table_pad(tag_table, W1[128:192], zb, 4096)
    dur1 = _prep_table_pad(duration_table, W1[192:256], zb, 2048)
    tim1 = _prep_table_pad(time_table, W1[256:320], zb, 2048)
    idx_small = (jnp.stack([
        categories.reshape(-1).astype(jnp.int32),
        tags.reshape(-1).astype(jnp.int32),
        dur_buckets.reshape(-1),
        time_buckets.reshape(-1),
    ]) * 2).reshape(4, _NW, ng, _G)
    vt_b, cat1, tag1, dur1, tim1 = jax.lax.optimization_barrier(
        (video_table, cat1, tag1, dur1, tim1))
    s_rows = _sc_gather_sum(
        idx_small,
        [a.reshape(a.shape[0] // _EMB, _EMB)
         for a in (cat1, tag1, dur1, tim1)], n_tok)
    vid1 = _prep_table_pad(vt_b, W1[0:64], zb, 8192)
    idx_video = (video_ids.reshape(-1).astype(jnp.int32)
                 * 2).reshape(1, _NW, ng, _G)
    (v_rows,) = _sc_gather(
        idx_video, [vid1.reshape(vid1.shape[0] // _EMB, _EMB)], n_tok)

    rows128 = [r.reshape(n_tok // 2, 2 * _EMB) for r in (v_rows, s_rows)]
    out = _tc_mlp(rows128, W2, b2, gamma, beta, n_tok // 2, blk=2048)
    return out.reshape(B, L, _EMB)


# speedup vs baseline: 1.1065x; 1.0308x over previous
import functools

import jax
import jax.numpy as jnp
from jax import lax
from jax.experimental import pallas as pl
from jax.experimental.pallas import tpu as pltpu
from jax.experimental.pallas import tpu_sc as plsc

_EMB = 64
_NC = 2
_NS = 16
_NW = _NC * _NS
_G = 128


def _prep_table_pad(table, w1s, bias, blk):
    V = table.shape[0]
    tT = table.T
    blk = min(V, blk)
    grid = ((V + blk - 1) // blk,)

    def body(x, w, b, o):
        y = x[...].T
        y = jnp.dot(y, w[...], preferred_element_type=jnp.float32) + b[...]
        z = jnp.concatenate([y, jnp.zeros((blk, _EMB), jnp.float32)], axis=1)
        o[...] = z.reshape(blk * 2 * _EMB)

    out = pl.pallas_call(
        body,
        grid=grid,
        in_specs=[pl.BlockSpec((_EMB, blk), lambda i: (0, i)),
                  pl.BlockSpec((_EMB, _EMB), lambda i: (0, 0)),
                  pl.BlockSpec((1, _EMB), lambda i: (0, 0))],
        out_specs=pl.BlockSpec((blk * 2 * _EMB,), lambda i: (i,)),
        out_shape=jax.ShapeDtypeStruct((V * 2 * _EMB,), jnp.float32),
    )(tT, w1s, bias)
    return out


def _sc_gather(idx2d, tables, n_tok):
    nt = len(tables)
    ng = n_tok // (_NW * _G)
    tpw = n_tok // _NW

    mesh = plsc.VectorSubcoreMesh(core_axis_name="c", subcore_axis_name="s",
                                  num_cores=_NC, num_subcores=_NS)

    @functools.partial(
        pl.kernel,
        out_type=tuple(jax.ShapeDtypeStruct((n_tok, _EMB), jnp.float32)
                       for _ in range(nt)),
        mesh=mesh,
        scratch_types=[
            pltpu.VMEM((nt, ng, _G), jnp.int32),
            pltpu.VMEM((nt, _G, _EMB), jnp.float32),
            pltpu.VMEM((nt, _G, _EMB), jnp.float32),
            pltpu.SemaphoreType.DMA,
            pltpu.SemaphoreType.DMA,
        ],
        compiler_params=pltpu.CompilerParams(use_tc_tiling_on_sc=False),
    )
    def k(idx_hbm, *rest):
        tabs = rest[:nt]
        outs = rest[nt:2 * nt]
        idx_v, rows_a, rows_b, sem_a, sem_b = rest[2 * nt:]
        wid = lax.axis_index("s") * _NC + lax.axis_index("c")
        tbase = wid * tpw
        for t in range(nt):
            pltpu.sync_copy(idx_hbm.at[t, wid], idx_v.at[t])

        def fire(g, buf, sem):
            return [pltpu.async_copy(tabs[t].at[idx_v.at[t, g]],
                                     buf.at[t], sem)
                    for t in range(nt)]

        fire(0, rows_a, sem_a)

        def pair2(i, carry):
            g0 = i * 2

            cps_b = fire(g0 + 1, rows_b, sem_b)
            for t in range(nt):
                pltpu.make_async_copy(tabs[t].at[idx_v.at[t, g0]],
                                      rows_a.at[t], sem_a).wait()
            for t in range(nt):
                pltpu.sync_copy(rows_a.at[t],
                                outs[t].at[pl.ds(tbase + g0 * _G, _G)])

            @pl.when(i < ng // 2 - 1)
            def _():
                fire(g0 + 2, rows_a, sem_a)

            for cp in cps_b:
                cp.wait()
            for t in range(nt):
                pltpu.sync_copy(rows_b.at[t],
                                outs[t].at[pl.ds(tbase + (g0 + 1) * _G, _G)])
            return carry

        lax.fori_loop(0, ng // 2, pair2, 0)

    return k(idx2d, *tables)


def _sc_gather_sum(idx2d, tables, n_tok):
    nt = len(tables)
    ng = n_tok // (_NW * _G)
    tpw = n_tok // _NW

    mesh = plsc.VectorSubcoreMesh(core_axis_name="c", subcore_axis_name="s",
                                  num_cores=_NC, num_subcores=_NS)

    @functools.partial(
        pl.kernel,
        out_type=jax.ShapeDtypeStruct((n_tok, _EMB), jnp.float32),
        mesh=mesh,
        scratch_types=[
            pltpu.VMEM((nt, ng, _G), jnp.int32),
            pltpu.VMEM((nt, _G, _EMB), jnp.float32),
            pltpu.VMEM((nt, _G, _EMB), jnp.float32),
            pltpu.VMEM((_G, _EMB), jnp.float32),
            pltpu.VMEM((_G, _EMB), jnp.float32),
            pltpu.SemaphoreType.DMA,
            pltpu.SemaphoreType.DMA,
        ],
        compiler_params=pltpu.CompilerParams(use_tc_tiling_on_sc=False),
    )
    def k(idx_hbm, *rest):
        tabs = rest[:nt]
        (out_s, idx_v, rows_a, rows_b, sum_a, sum_b,
         sem_a, sem_b) = rest[nt:]
        wid = lax.axis_index("s") * _NC + lax.axis_index("c")
        tbase = wid * tpw
        for t in range(nt):
            pltpu.sync_copy(idx_hbm.at[t, wid], idx_v.at[t])

        def fire(g, buf, sem):
            return [pltpu.async_copy(tabs[t].at[idx_v.at[t, g]],
                                     buf.at[t], sem)
                    for t in range(nt)]

        def drain(g, buf, sem):
            for t in range(nt):
                pltpu.make_async_copy(tabs[t].at[idx_v.at[t, g]],
                                      buf.at[t], sem).wait()

        def vsum(buf, sbuf):
            def add1(v, carry):
                r = v // 4
                c = pl.multiple_of((v % 4) * 16, 16)
                x = buf[0, r, pl.ds(c, 16)]
                for t in range(1, nt):
                    x = x + buf[t, r, pl.ds(c, 16)]
                sbuf[r, pl.ds(c, 16)] = x
                return carry
            lax.fori_loop(0, _G * 4, add1, 0)

        fire(0, rows_a, sem_a)

        def pair2(i, carry):
            g0 = i * 2
            cps_b = fire(g0 + 1, rows_b, sem_b)
            drain(g0, rows_a, sem_a)
            vsum(rows_a, sum_a)

            @pl.when(i < ng // 2 - 1)
            def _():
                fire(g0 + 2, rows_a, sem_a)

            pltpu.sync_copy(sum_a, out_s.at[pl.ds(tbase + g0 * _G, _G)])
            for cp in cps_b:
                cp.wait()
            vsum(rows_b, sum_b)
            pltpu.sync_copy(sum_b, out_s.at[pl.ds(tbase + (g0 + 1) * _G, _G)])
            return carry

        lax.fori_loop(0, ng // 2, pair2, 0)

    return k(idx2d, *tables)


def _tc_mlp(rows128, W2, b2, gamma, beta, n_half, blk):
    grid = (n_half // blk,)
    nr = len(rows128)

    def half(h, w2, b2r, gm, bt):
        h2 = jnp.dot(h, w2, preferred_element_type=jnp.float32) + b2r
        mu = jnp.mean(h2, axis=-1, keepdims=True)
        var = jnp.mean((h2 - mu) ** 2, axis=-1, keepdims=True)
        return (h2 - mu) / jnp.sqrt(var + 1e-3) * gm + bt

    def body(*refs):
        xs = refs[:nr]
        w2, b2r, gm, bt, o = refs[nr:]
        s = xs[0][...]
        for x in xs[1:]:
            s = s + x[...]
        h = jnp.maximum(s, 0.0)
        args = (w2[...], b2r[...], gm[...], bt[...])
        re = half(h[:, 0:64], *args)
        ro = half(h[:, 64:128], *args)
        o[...] = jnp.stack([re, ro], axis=1).reshape(2 * blk, _EMB)

    tok_spec = pl.BlockSpec((blk, 2 * _EMB), lambda i: (i, 0))
    out_spec = pl.BlockSpec((2 * blk, _EMB), lambda i: (i, 0))
    full = lambda shape: pl.BlockSpec(shape, lambda i: tuple(0 for _ in shape))
    return pl.pallas_call(
        body,
        grid=grid,
        in_specs=[tok_spec] * nr + [
            full((_EMB, _EMB)), full((1, _EMB)),
            full((1, _EMB)), full((1, _EMB)),
        ],
        out_specs=out_spec,
        out_shape=jax.ShapeDtypeStruct((2 * n_half, _EMB), jnp.float32),
    )(*rows128, W2, b2.reshape(1, _EMB),
      gamma.reshape(1, _EMB), beta.reshape(1, _EMB))


def kernel(video_ids, categories, tags, durations, timestamps,
           video_table, category_table, tag_table, duration_table, time_table,
           W1, b1, W2, b2, gamma, beta):
    B, L = video_ids.shape
    n_tok = B * L
    ng = n_tok // (_NW * _G)
    dur_buckets = (durations / 300.0 * 100.0).astype(jnp.int32)
    time_buckets = (timestamps % 168).astype(jnp.int32)
    zb = jnp.zeros((1, _EMB), jnp.float32)
    cat1 = _prep_table_pad(category_table, W1[64:128], b1.reshape(1, _EMB),
                           2048)
    tag1 = _prep_table_pad(tag_table, W1[128:192], zb, 4096)
    dur1 = _prep_table_pad(duration_table, W1[192:256], zb, 2048)
    tim1 = _prep_table_pad(time_table, W1[256:320], zb, 2048)
    idx_small = (jnp.stack([
        categories.reshape(-1).astype(jnp.int32),
        tags.reshape(-1).astype(jnp.int32),
        dur_buckets.reshape(-1),
        time_buckets.reshape(-1),
    ]) * 2).reshape(4, _NW, ng, _G)
    vt_b, cat1, tag1, dur1, tim1 = jax.lax.optimization_barrier(
        (video_table, cat1, tag1, dur1, tim1))
    s_rows = _sc_gather_sum(
        idx_small,
        [a.reshape(a.shape[0] // _EMB, _EMB)
         for a in (cat1, tag1, dur1, tim1)], n_tok)
    vid1 = _prep_table_pad(vt_b, W1[0:64], zb, 16384)
    idx_video = (video_ids.reshape(-1).astype(jnp.int32)
                 * 2).reshape(1, _NW, ng, _G)
    (v_rows,) = _sc_gather(
        idx_video, [vid1.reshape(vid1.shape[0] // _EMB, _EMB)], n_tok)

    rows128 = [r.reshape(n_tok // 2, 2 * _EMB) for r in (v_rows, s_rows)]
    out = _tc_mlp(rows128, W2, b2, gamma, beta, n_tok // 2, blk=4096)
    return out.reshape(B, L, _EMB)


# speedup vs baseline: 1.1257x; 1.0173x over previous
import functools

import jax
import jax.numpy as jnp
from jax import lax
from jax.experimental import pallas as pl
from jax.experimental.pallas import tpu as pltpu
from jax.experimental.pallas import tpu_sc as plsc

_EMB = 64
_NC = 2
_NS = 16
_NW = _NC * _NS
_G = 128


def _prep_table_pad(table, w1s, bias, blk):
    V = table.shape[0]
    tT = table.T
    blk = min(V, blk)
    grid = ((V + blk - 1) // blk,)

    def body(x, w, b, o):
        y = x[...].T
        y = jnp.dot(y, w[...], preferred_element_type=jnp.float32) + b[...]
        z = jnp.concatenate([y, jnp.zeros((blk, _EMB), jnp.float32)], axis=1)
        o[...] = z.reshape(blk * 2 * _EMB)

    out = pl.pallas_call(
        body,
        grid=grid,
        in_specs=[pl.BlockSpec((_EMB, blk), lambda i: (0, i)),
                  pl.BlockSpec((_EMB, _EMB), lambda i: (0, 0)),
                  pl.BlockSpec((1, _EMB), lambda i: (0, 0))],
        out_specs=pl.BlockSpec((blk * 2 * _EMB,), lambda i: (i,)),
        out_shape=jax.ShapeDtypeStruct((V * 2 * _EMB,), jnp.float32),
    )(tT, w1s, bias)
    return out


def _sc_gather(idx2d, tables, n_tok):
    nt = len(tables)
    ng = n_tok // (_NW * _G)
    tpw = n_tok // _NW

    mesh = plsc.VectorSubcoreMesh(core_axis_name="c", subcore_axis_name="s",
                                  num_cores=_NC, num_subcores=_NS)

    @functools.partial(
        pl.kernel,
        out_type=tuple(jax.ShapeDtypeStruct((n_tok, _EMB), jnp.float32)
                       for _ in range(nt)),
        mesh=mesh,
        scratch_types=[
            pltpu.VMEM((nt, ng, _G), jnp.int32),
            pltpu.VMEM((nt, _G, _EMB), jnp.float32),
            pltpu.VMEM((nt, _G, _EMB), jnp.float32),
            pltpu.SemaphoreType.DMA,
            pltpu.SemaphoreType.DMA,
        ],
        compiler_params=pltpu.CompilerParams(use_tc_tiling_on_sc=False),
    )
    def k(idx_hbm, *rest):
        tabs = rest[:nt]
        outs = rest[nt:2 * nt]
        idx_v, rows_a, rows_b, sem_a, sem_b = rest[2 * nt:]
        wid = lax.axis_index("s") * _NC + lax.axis_index("c")
        tbase = wid * tpw
        for t in range(nt):
            pltpu.sync_copy(idx_hbm.at[t, wid], idx_v.at[t])

        def fire(g, buf, sem):
            return [pltpu.async_copy(tabs[t].at[idx_v.at[t, g]],
                                     buf.at[t], sem)
                    for t in range(nt)]

        fire(0, rows_a, sem_a)

        def pair2(i, carry):
            g0 = i * 2

            cps_b = fire(g0 + 1, rows_b, sem_b)
            for t in range(nt):
                pltpu.make_async_copy(tabs[t].at[idx_v.at[t, g0]],
                                      rows_a.at[t], sem_a).wait()
            for t in range(nt):
                pltpu.sync_copy(rows_a.at[t],
                                outs[t].at[pl.ds(tbase + g0 * _G, _G)])

            @pl.when(i < ng // 2 - 1)
            def _():
                fire(g0 + 2, rows_a, sem_a)

            for cp in cps_b:
                cp.wait()
            for t in range(nt):
                pltpu.sync_copy(rows_b.at[t],
                                outs[t].at[pl.ds(tbase + (g0 + 1) * _G, _G)])
            return carry

        lax.fori_loop(0, ng // 2, pair2, 0)

    return k(idx2d, *tables)


def _sc_gather_sum(idx2d, tables, n_tok):
    nt = len(tables)
    ng = n_tok // (_NW * _G)
    tpw = n_tok // _NW

    mesh = plsc.VectorSubcoreMesh(core_axis_name="c", subcore_axis_name="s",
                                  num_cores=_NC, num_subcores=_NS)

    @functools.partial(
        pl.kernel,
        out_type=jax.ShapeDtypeStruct((n_tok, _EMB), jnp.float32),
        mesh=mesh,
        scratch_types=[
            pltpu.VMEM((nt, ng, _G), jnp.int32),
            pltpu.VMEM((nt, _G, _EMB), jnp.float32),
            pltpu.VMEM((nt, _G, _EMB), jnp.float32),
            pltpu.VMEM((_G, _EMB), jnp.float32),
            pltpu.VMEM((_G, _EMB), jnp.float32),
            pltpu.SemaphoreType.DMA,
            pltpu.SemaphoreType.DMA,
        ],
        compiler_params=pltpu.CompilerParams(use_tc_tiling_on_sc=False),
    )
    def k(idx_hbm, *rest):
        tabs = rest[:nt]
        (out_s, idx_v, rows_a, rows_b, sum_a, sum_b,
         sem_a, sem_b) = rest[nt:]
        wid = lax.axis_index("s") * _NC + lax.axis_index("c")
        tbase = wid * tpw
        for t in range(nt):
            pltpu.sync_copy(idx_hbm.at[t, wid], idx_v.at[t])

        def fire(g, buf, sem):
            return [pltpu.async_copy(tabs[t].at[idx_v.at[t, g]],
                                     buf.at[t], sem)
                    for t in range(nt)]

        def drain(g, buf, sem):
            for t in range(nt):
                pltpu.make_async_copy(tabs[t].at[idx_v.at[t, g]],
                                      buf.at[t], sem).wait()

        def vsum(buf, sbuf):
            def add1(v, carry):
                r = v // 4
                c = pl.multiple_of((v % 4) * 16, 16)
                x = buf[0, r, pl.ds(c, 16)]
                for t in range(1, nt):
                    x = x + buf[t, r, pl.ds(c, 16)]
                sbuf[r, pl.ds(c, 16)] = x
                return carry
            lax.fori_loop(0, _G * 4, add1, 0)

        fire(0, rows_a, sem_a)

        def pair2(i, carry):
            g0 = i * 2
            cps_b = fire(g0 + 1, rows_b, sem_b)
            drain(g0, rows_a, sem_a)
            vsum(rows_a, sum_a)

            @pl.when(i < ng // 2 - 1)
            def _():
                fire(g0 + 2, rows_a, sem_a)

            pltpu.sync_copy(sum_a, out_s.at[pl.ds(tbase + g0 * _G, _G)])
            for cp in cps_b:
                cp.wait()
            vsum(rows_b, sum_b)
            pltpu.sync_copy(sum_b, out_s.at[pl.ds(tbase + (g0 + 1) * _G, _G)])
            return carry

        lax.fori_loop(0, ng // 2, pair2, 0)

    return k(idx2d, *tables)


def _tc_mlp(rows128, W2, b2, gamma, beta, n_half, blk):
    grid = (n_half // blk,)
    nr = len(rows128)

    def half(h, w2, b2r, gm, bt):
        h2 = jnp.dot(h, w2, preferred_element_type=jnp.float32) + b2r
        mu = jnp.mean(h2, axis=-1, keepdims=True)
        var = jnp.mean((h2 - mu) ** 2, axis=-1, keepdims=True)
        return (h2 - mu) / jnp.sqrt(var + 1e-3) * gm + bt

    def body(*refs):
        xs = refs[:nr]
        w2, b2r, gm, bt, o = refs[nr:]
        s = xs[0][...]
        for x in xs[1:]:
            s = s + x[...]
        h = jnp.maximum(s, 0.0)
        args = (w2[...], b2r[...], gm[...], bt[...])
        re = half(h[:, 0:64], *args)
        ro = half(h[:, 64:128], *args)
        o[...] = jnp.stack([re, ro], axis=1).reshape(2 * blk, _EMB)

    tok_spec = pl.BlockSpec((blk, 2 * _EMB), lambda i: (i, 0))
    out_spec = pl.BlockSpec((2 * blk, _EMB), lambda i: (i, 0))
    full = lambda shape: pl.BlockSpec(shape, lambda i: tuple(0 for _ in shape))
    return pl.pallas_call(
        body,
        grid=grid,
        in_specs=[tok_spec] * nr + [
            full((_EMB, _EMB)), full((1, _EMB)),
            full((1, _EMB)), full((1, _EMB)),
        ],
        out_specs=out_spec,
        out_shape=jax.ShapeDtypeStruct((2 * n_half, _EMB), jnp.float32),
    )(*rows128, W2, b2.reshape(1, _EMB),
      gamma.reshape(1, _EMB), beta.reshape(1, _EMB))


def kernel(video_ids, categories, tags, durations, timestamps,
           video_table, category_table, tag_table, duration_table, time_table,
           W1, b1, W2, b2, gamma, beta):
    B, L = video_ids.shape
    n_tok = B * L
    ng = n_tok // (_NW * _G)
    dur_buckets = (durations / 300.0 * 100.0).astype(jnp.int32)
    time_buckets = (timestamps % 168).astype(jnp.int32)
    zb = jnp.zeros((1, _EMB), jnp.float32)
    cat1 = _prep_table_pad(category_table, W1[64:128], b1.reshape(1, _EMB),
                           2048)
    tag1 = _prep_table_pad(tag_table, W1[128:192], zb, 4096)
    dur1 = _prep_table_pad(duration_table, W1[192:256], zb, 2048)
    tim1 = _prep_table_pad(time_table, W1[256:320], zb, 2048)
    idx_small = (jnp.stack([
        categories.reshape(-1).astype(jnp.int32),
        tags.reshape(-1).astype(jnp.int32),
        dur_buckets.reshape(-1),
        time_buckets.reshape(-1),
    ]) * 2).reshape(4, _NW, ng, _G)
    vt_b, cat1, tag1, dur1, tim1 = jax.lax.optimization_barrier(
        (video_table, cat1, tag1, dur1, tim1))
    s_rows = _sc_gather_sum(
        idx_small,
        [a.reshape(a.shape[0] // _EMB, _EMB)
         for a in (cat1, tag1, dur1, tim1)], n_tok)
    vid1 = _prep_table_pad(vt_b, W1[0:64], zb, 32768)
    idx_video = (video_ids.reshape(-1).astype(jnp.int32)
                 * 2).reshape(1, _NW, ng, _G)
    (v_rows,) = _sc_gather(
        idx_video, [vid1.reshape(vid1.shape[0] // _EMB, _EMB)], n_tok)

    rows128 = [r.reshape(n_tok // 2, 2 * _EMB) for r in (v_rows, s_rows)]
    out = _tc_mlp(rows128, W2, b2, gamma, beta, n_tok // 2, blk=8192)
    return out.reshape(B, L, _EMB)
